# Initial kernel scaffold; baseline (speedup 1.0000x reference)
import jax
import jax.numpy as jnp
from jax.experimental import pallas as pl


def kernel(unknown, known2, feats2, known3, feats3, known4, feats4, match_points, W_fc, W_cls):
    raise NotImplementedError("write your pallas kernel here")


# speedup vs baseline: 11.9378x; 11.9378x over previous
import jax
import jax.numpy as jnp
from jax import lax
from jax.experimental import pallas as pl
from jax.experimental.pallas import tpu as pltpu

_B = 256
_CHUNK = 2048
_INF = 3.0e38


def _argmin3_payload(A, G, iota):
    W = A.shape[1]
    ms, gs = [], []
    for _ in range(3):
        m = jnp.min(A, axis=1, keepdims=True)
        eq = A == m
        first = jnp.min(jnp.where(eq, iota, W), axis=1, keepdims=True)
        sel = iota == first
        g = jnp.sum(jnp.where(sel, G, 0.0), axis=1, keepdims=True)
        A = jnp.where(sel, _INF, A)
        ms.append(m)
        gs.append(g)
    return ms, gs


def _body(u_ref, k2_ref, k3_ref, k4_ref, mt_ref, f2_ref, f3_ref, f4_ref,
          wfc_ref, wcls_ref, pred_ref, gt_ref):
    u = u_ref[...]
    a2 = jnp.sum(u * u, axis=1, keepdims=True)
    vmat = jnp.dot(wcls_ref[...], wfc_ref[...])

    pred = jnp.zeros((_B, 1), jnp.float32)
    for kref, fref, off in ((k2_ref, f2_ref, 0),
                            (k3_ref, f3_ref, 32),
                            (k4_ref, f4_ref, 64)):
        kT = kref[...]
        M = kT.shape[1]
        b2 = jnp.sum(kT * kT, axis=0, keepdims=True)
        gfull = jnp.dot(vmat[:, off:off + 32], fref[...])
        G = gfull[0:1, :]
        cand_m, cand_g = [], []
        for c0 in range(0, M, _CHUNK):
            W = min(_CHUNK, M - c0)
            A = (a2 + b2[:, c0:c0 + W]) - 2.0 * jnp.dot(u, kT[:, c0:c0 + W])
            A = jnp.maximum(A, 0.0)
            iota = lax.broadcasted_iota(jnp.int32, (_B, W), 1)
            ms, gs = _argmin3_payload(A, G[:, c0:c0 + W], iota)
            cand_m += ms
            cand_g += gs
        if len(cand_m) > 3:
            Ac = jnp.concatenate(cand_m, axis=1)
            Gc = jnp.concatenate(cand_g, axis=1)
            iota = lax.broadcasted_iota(jnp.int32, Ac.shape, 1)
            ms, gs = _argmin3_payload(Ac, Gc, iota)
        d = [jnp.sqrt(m) for m in ms]
        r = [1.0 / (dd + 1e-8) for dd in d]
        pred = pred + (r[0] * gs[0] + r[1] * gs[1] + r[2] * gs[2]) / (r[0] + r[1] + r[2])
    pred_ref[...] = pred

    mt = mt_ref[...]
    b2m = jnp.sum(mt * mt, axis=0, keepdims=True)
    Am = jnp.maximum((a2 + b2m) - 2.0 * jnp.dot(u, mt), 0.0)
    min_d = jnp.sqrt(jnp.min(Am, axis=1, keepdims=True))
    gt_ref[...] = (min_d < 0.5).astype(jnp.float32)


def kernel(unknown, known2, feats2, known3, feats3, known4, feats4,
           match_points, W_fc, W_cls):
    N = unknown.shape[0]

    def pad_t(pts):
        return jnp.zeros((8, pts.shape[0]), jnp.float32).at[:3, :].set(pts.T)

    upad = jnp.zeros((N, 8), jnp.float32).at[:, :3].set(unknown)
    k2, k3, k4, mt = pad_t(known2), pad_t(known3), pad_t(known4), pad_t(match_points)
    f2, f3, f4 = feats2.T, feats3.T, feats4.T
    wcls = jnp.zeros((8, 64), jnp.float32).at[0:1, :].set(W_cls)

    full = lambda shape: pl.BlockSpec(shape, lambda i: (0, 0))
    pred, gt = pl.pallas_call(
        _body,
        grid=(N // _B,),
        in_specs=[
            pl.BlockSpec((_B, 8), lambda i: (i, 0)),
            full(k2.shape), full(k3.shape), full(k4.shape), full(mt.shape),
            full(f2.shape), full(f3.shape), full(f4.shape),
            full(W_fc.shape), full(wcls.shape),
        ],
        out_specs=[
            pl.BlockSpec((_B, 1), lambda i: (i, 0)),
            pl.BlockSpec((_B, 1), lambda i: (i, 0)),
        ],
        out_shape=[
            jax.ShapeDtypeStruct((N, 1), jnp.float32),
            jax.ShapeDtypeStruct((N, 1), jnp.float32),
        ],
        compiler_params=pltpu.CompilerParams(dimension_semantics=("arbitrary",)),
    )(upad, k2, k3, k4, mt, f2, f3, f4, W_fc, wcls)
    return pred, gt.reshape(N)


# speedup vs baseline: 14.6299x; 1.2255x over previous
import jax
import jax.numpy as jnp
from jax import lax
from jax.experimental import pallas as pl
from jax.experimental.pallas import tpu as pltpu

_B = 256
_CHUNK = 2048
_INF = 3.0e38
_BIG = 1.0e9


def _argmin3_payload(A, G, iotaf):
    ms, gs = [], []
    for rnd in range(3):
        m = jnp.min(A, axis=1, keepdims=True)
        idx = jnp.where(A == m, iotaf, _BIG)
        c = jnp.min(idx, axis=1, keepdims=True)
        sel = idx == c
        g = jnp.sum(jnp.where(sel, G, 0.0), axis=1, keepdims=True)
        if rnd < 2:
            A = jnp.where(sel, _INF, A)
        ms.append(m)
        gs.append(g)
    return ms, gs


def _body(u_ref, k2_ref, k3_ref, k4_ref, mt_ref, f2_ref, f3_ref, f4_ref,
          wfc_ref, wcls_ref, pred_ref, gt_ref):
    u = u_ref[...]
    a2 = jnp.sum(u * u, axis=1, keepdims=True)
    un2 = u * -2.0
    vmat = jnp.dot(wcls_ref[...], wfc_ref[...])

    pred = jnp.zeros((_B, 1), jnp.float32)
    for kref, fref, off in ((k2_ref, f2_ref, 0),
                            (k3_ref, f3_ref, 32),
                            (k4_ref, f4_ref, 64)):
        kT = kref[...]
        M = kT.shape[1]
        b2 = jnp.sum(kT * kT, axis=0, keepdims=True)
        gfull = jnp.dot(vmat[:, off:off + 32], fref[...])
        G = gfull[0:1, :]
        cand_m, cand_g = [], []
        for c0 in range(0, M, _CHUNK):
            W = min(_CHUNK, M - c0)
            A = (a2 + b2[:, c0:c0 + W]) + jnp.dot(un2, kT[:, c0:c0 + W])
            A = jnp.maximum(A, 0.0)
            iotaf = lax.broadcasted_iota(jnp.int32, (_B, W), 1).astype(jnp.float32)
            ms, gs = _argmin3_payload(A, G[:, c0:c0 + W], iotaf)
            cand_m += ms
            cand_g += gs
        if len(cand_m) > 3:
            Ac = jnp.concatenate(cand_m, axis=1)
            Gc = jnp.concatenate(cand_g, axis=1)
            iotaf = lax.broadcasted_iota(jnp.int32, Ac.shape, 1).astype(jnp.float32)
            ms, gs = _argmin3_payload(Ac, Gc, iotaf)
        d = [jnp.sqrt(m) for m in ms]
        r = [1.0 / (dd + 1e-8) for dd in d]
        pred = pred + (r[0] * gs[0] + r[1] * gs[1] + r[2] * gs[2]) / (r[0] + r[1] + r[2])
    pred_ref[...] = pred

    mt = mt_ref[...]
    b2m = jnp.sum(mt * mt, axis=0, keepdims=True)
    Am = jnp.maximum((a2 + b2m) + jnp.dot(un2, mt), 0.0)
    min_d = jnp.sqrt(jnp.min(Am, axis=1, keepdims=True))
    gt_ref[...] = (min_d < 0.5).astype(jnp.float32)


def kernel(unknown, known2, feats2, known3, feats3, known4, feats4,
           match_points, W_fc, W_cls):
    N = unknown.shape[0]

    def pad_t(pts):
        return jnp.zeros((8, pts.shape[0]), jnp.float32).at[:3, :].set(pts.T)

    upad = jnp.zeros((N, 8), jnp.float32).at[:, :3].set(unknown)
    k2, k3, k4, mt = pad_t(known2), pad_t(known3), pad_t(known4), pad_t(match_points)
    f2, f3, f4 = feats2.T, feats3.T, feats4.T
    wcls = jnp.zeros((8, 64), jnp.float32).at[0:1, :].set(W_cls)

    full = lambda shape: pl.BlockSpec(shape, lambda i: (0, 0))
    pred, gt = pl.pallas_call(
        _body,
        grid=(N // _B,),
        in_specs=[
            pl.BlockSpec((_B, 8), lambda i: (i, 0)),
            full(k2.shape), full(k3.shape), full(k4.shape), full(mt.shape),
            full(f2.shape), full(f3.shape), full(f4.shape),
            full(W_fc.shape), full(wcls.shape),
        ],
        out_specs=[
            pl.BlockSpec((_B, 1), lambda i: (i, 0)),
            pl.BlockSpec((_B, 1), lambda i: (i, 0)),
        ],
        out_shape=[
            jax.ShapeDtypeStruct((N, 1), jnp.float32),
            jax.ShapeDtypeStruct((N, 1), jnp.float32),
        ],
        compiler_params=pltpu.CompilerParams(dimension_semantics=("arbitrary",)),
    )(upad, k2, k3, k4, mt, f2, f3, f4, W_fc, wcls)
    return pred, gt.reshape(N)


# speedup vs baseline: 16.0863x; 1.0996x over previous
import functools

import jax
import jax.numpy as jnp
from jax import lax
from jax.experimental import pallas as pl
from jax.experimental.pallas import tpu as pltpu
from jax.experimental.pallas import tpu_sc as plsc

_B = 256
_CHUNK = 2048
_INF = 3.0e38
_BIG = 1.0e9

_N = 8192
_SIZES = (8192, 4096, 2048)
_GOFF = (0, 8192, 12288)
_GTOT = 14336

_NC, _NS, _L = 2, 16, 16
_NW = _NC * _NS
_RPW = _N // _NW


def _argmin3(A, iotaf, want_mask_last=False):
    ms, cs = [], []
    for rnd in range(3):
        m = jnp.min(A, axis=1, keepdims=True)
        idx = jnp.where(A == m, iotaf, _BIG)
        c = jnp.min(idx, axis=1, keepdims=True)
        if rnd < 2:
            A = jnp.where(idx == c, _INF, A)
        ms.append(m)
        cs.append(c)
    return ms, cs


def _argmin3_payload(A, G, iotaf):
    ms, gs = [], []
    for rnd in range(3):
        m = jnp.min(A, axis=1, keepdims=True)
        idx = jnp.where(A == m, iotaf, _BIG)
        c = jnp.min(idx, axis=1, keepdims=True)
        sel = idx == c
        g = jnp.sum(jnp.where(sel, G, 0.0), axis=1, keepdims=True)
        if rnd < 2:
            A = jnp.where(sel, _INF, A)
        ms.append(m)
        gs.append(g)
    return ms, gs


def _tc_body(u_ref, k2_ref, k3_ref, k4_ref, mt_ref, f2_ref, f3_ref, f4_ref,
             wfc_ref, wcls_ref, w_ref, i_ref, g_ref, gt_ref):
    pid = pl.program_id(0)
    u = u_ref[...]
    a2 = jnp.sum(u * u, axis=1, keepdims=True)
    un2 = u * -2.0
    vmat = jnp.dot(wcls_ref[...], wfc_ref[...])

    ws, idxs = [], []
    for s, (kref, fref) in enumerate(((k2_ref, f2_ref), (k3_ref, f3_ref),
                                      (k4_ref, f4_ref))):
        kT = kref[...]
        M = kT.shape[1]
        b2 = jnp.sum(kT * kT, axis=0, keepdims=True)

        @pl.when(pid == 0)
        def _write_g():
            gfull = jnp.dot(vmat[:, 32 * s:32 * s + 32], fref[...])
            g_ref[0:1, _GOFF[s]:_GOFF[s] + M] = gfull[0:1, :]

        cand_m, cand_i = [], []
        for c0 in range(0, M, _CHUNK):
            W = min(_CHUNK, M - c0)
            A = (a2 + b2[:, c0:c0 + W]) + jnp.dot(un2, kT[:, c0:c0 + W])
            A = jnp.maximum(A, 0.0)
            iotaf = lax.broadcasted_iota(jnp.int32, (_B, W), 1).astype(jnp.float32)
            ms, cs = _argmin3(A, iotaf)
            cand_m += ms
            cand_i += [c + float(c0 + _GOFF[s]) for c in cs]
        if len(cand_m) > 3:
            Ac = jnp.concatenate(cand_m, axis=1)
            Ic = jnp.concatenate(cand_i, axis=1)
            iotaf = lax.broadcasted_iota(jnp.int32, Ac.shape, 1).astype(jnp.float32)
            ms, gidx = _argmin3_payload(Ac, Ic, iotaf)
        else:
            ms, gidx = cand_m, cand_i
        d = [jnp.sqrt(m) for m in ms]
        r = [1.0 / (dd + 1e-8) for dd in d]
        norm = r[0] + r[1] + r[2]
        ws += [rj / norm for rj in r]
        idxs += gidx

    zero7 = jnp.zeros((_B, 7), jnp.float32)
    w_ref[...] = jnp.concatenate(ws + [zero7], axis=1)
    i_ref[...] = jnp.concatenate(idxs + [zero7], axis=1).astype(jnp.int32)

    mt = mt_ref[...]
    b2m = jnp.sum(mt * mt, axis=0, keepdims=True)
    Am = jnp.maximum((a2 + b2m) + jnp.dot(un2, mt), 0.0)
    min_d = jnp.sqrt(jnp.min(Am, axis=1, keepdims=True))
    gt_ref[...] = (min_d < 0.5).astype(jnp.float32)


def _sc_body(w_hbm, i_hbm, g_hbm, out_hbm, wv, iv, gath, acc, sem):
    wid = lax.axis_index("s") * _NC + lax.axis_index("c")
    base = wid * _RPW
    copies = []
    for j in range(9):
        copies.append(pltpu.async_copy(
            w_hbm.at[pl.ds(j * _N + base, _RPW)], wv.at[pl.ds(j * _RPW, _RPW)], sem))
        copies.append(pltpu.async_copy(
            i_hbm.at[pl.ds(j * _N + base, _RPW)], iv.at[pl.ds(j * _RPW, _RPW)], sem))
    for cp in copies:
        cp.wait()
    pltpu.async_copy(g_hbm.at[iv], gath, sem).wait()
    for t in range(_RPW // _L):
        a = jnp.zeros((_L,), jnp.float32)
        for j in range(9):
            o = j * _RPW + t * _L
            a = a + wv[pl.ds(o, _L)] * gath[pl.ds(o, _L)]
        acc[pl.ds(t * _L, _L)] = a
    pltpu.sync_copy(acc, out_hbm.at[pl.ds(base, _RPW)])


def kernel(unknown, known2, feats2, known3, feats3, known4, feats4,
           match_points, W_fc, W_cls):
    N = unknown.shape[0]

    def pad_t(pts):
        return jnp.zeros((8, pts.shape[0]), jnp.float32).at[:3, :].set(pts.T)

    upad = jnp.zeros((N, 8), jnp.float32).at[:, :3].set(unknown)
    k2, k3, k4, mt = pad_t(known2), pad_t(known3), pad_t(known4), pad_t(match_points)
    f2, f3, f4 = feats2.T, feats3.T, feats4.T
    wcls = jnp.zeros((8, 64), jnp.float32).at[0:1, :].set(W_cls)

    full = lambda shape: pl.BlockSpec(shape, lambda i: (0, 0))
    w9, i9, gcat, gt = pl.pallas_call(
        _tc_body,
        grid=(N // _B,),
        in_specs=[
            pl.BlockSpec((_B, 8), lambda i: (i, 0)),
            full(k2.shape), full(k3.shape), full(k4.shape), full(mt.shape),
            full(f2.shape), full(f3.shape), full(f4.shape),
            full(W_fc.shape), full(wcls.shape),
        ],
        out_specs=[
            pl.BlockSpec((_B, 16), lambda i: (i, 0)),
            pl.BlockSpec((_B, 16), lambda i: (i, 0)),
            pl.BlockSpec((1, _GTOT), lambda i: (0, 0)),
            pl.BlockSpec((_B, 1), lambda i: (i, 0)),
        ],
        out_shape=[
            jax.ShapeDtypeStruct((N, 16), jnp.float32),
            jax.ShapeDtypeStruct((N, 16), jnp.int32),
            jax.ShapeDtypeStruct((1, _GTOT), jnp.float32),
            jax.ShapeDtypeStruct((N, 1), jnp.float32),
        ],
        compiler_params=pltpu.CompilerParams(dimension_semantics=("arbitrary",)),
    )(upad, k2, k3, k4, mt, f2, f3, f4, W_fc, wcls)

    pred = _sc_combine(w9.T[:9].reshape(-1), i9.T[:9].reshape(-1), gcat.reshape(_GTOT))
    return pred.reshape(N, 1), gt.reshape(N)


def _sc_combine(w9, i9, gcat):
    sc_call = functools.partial(
        pl.kernel,
        out_type=jax.ShapeDtypeStruct((_N,), jnp.float32),
        mesh=plsc.VectorSubcoreMesh(core_axis_name="c", subcore_axis_name="s"),
        scratch_types=[
            pltpu.VMEM((9 * _RPW,), jnp.float32),
            pltpu.VMEM((9 * _RPW,), jnp.int32),
            pltpu.VMEM((9 * _RPW,), jnp.float32),
            pltpu.VMEM((_RPW,), jnp.float32),
            pltpu.SemaphoreType.DMA,
        ],
    )(_sc_body)
    return sc_call(w9, i9, gcat)
